# baseline (speedup 1.0000x reference)
import jax
import jax.numpy as jnp
from jax import lax
from jax.experimental import pallas as pl
from jax.experimental.pallas import tpu as pltpu
from jax.experimental.pallas import tpu_sc as plsc

NUM_TOKENS = 32768
HID = 32
NE = 4
TOPK = 2

NC = 2
NS = 16
L = 16
NW = NC * NS
CHUNK = NUM_TOKENS // NW
GROUPS = CHUNK // L
HPAIR = HID // 2

_mesh = plsc.VectorSubcoreMesh(
    core_axis_name="c", subcore_axis_name="s", num_cores=NC, num_subcores=NS
)

_OUT_TYPE = (
    jax.ShapeDtypeStruct((NUM_TOKENS * NE,), jnp.float32),
    jax.ShapeDtypeStruct((NUM_TOKENS * TOPK,), jnp.float32),
    jax.ShapeDtypeStruct((NUM_TOKENS * TOPK,), jnp.int32),
)
_SCRATCH = [
    pltpu.VMEM((CHUNK * HPAIR,), jnp.int32),
    pltpu.VMEM((NE * HID,), jnp.float32),
    pltpu.VMEM((CHUNK * NE,), jnp.float32),
    pltpu.VMEM((CHUNK * TOPK,), jnp.float32),
    pltpu.VMEM((CHUNK * TOPK,), jnp.int32),
]


def _tree_sum(vs):
    while len(vs) > 1:
        vs = [vs[i] + vs[i + 1] for i in range(0, len(vs) - 1, 2)] + (
            [vs[-1]] if len(vs) % 2 else []
        )
    return vs[0]


def _router_body(hs_hbm, w_hbm, probs_hbm, tv_hbm, ti_hbm, x_v, w_v, p_v, tv_v, ti_v):
    wid = lax.axis_index("s") * NC + lax.axis_index("c")
    base = wid * CHUNK

    pltpu.sync_copy(w_hbm, w_v)
    pltpu.sync_copy(hs_hbm.at[pl.ds(base * HPAIR, CHUNK * HPAIR)], x_v)

    w = []
    for e in range(NE):
        halves = [w_v[pl.ds(e * HID, L)], w_v[pl.ds(e * HID + L, L)]]
        w.append([halves[d // L][d % L] for d in range(HID)])

    lane = lax.iota(jnp.int32, L)
    lane_h = lane * HPAIR
    lane_e = lane * NE
    lane_k = lane * TOPK

    def body(g, carry):
        rowb = lane_h + g * (L * HPAIR)
        cols = []
        for j in range(HPAIR):
            pair = plsc.load_gather(x_v, [rowb + j])
            cols.extend(
                plsc.unpack(
                    plsc.bitcast(pair, jnp.bfloat16),
                    format=plsc.PackFormat.INTERLEAVED,
                )
            )

        logits = [_tree_sum([cols[d] * w[e][d] for d in range(HID)]) for e in range(NE)]

        m = jnp.maximum(
            jnp.maximum(logits[0], logits[1]), jnp.maximum(logits[2], logits[3])
        )
        ex = [jnp.exp(a - m) for a in logits]
        r = 1.0 / _tree_sum(list(ex))
        p = [e_ * r for e_ in ex]

        pb = lane_e + g * (L * NE)
        for e in range(NE):
            plsc.store_scatter(p_v, [pb + e], p[e])

        b1l, b1v, b1i = logits[0], p[0], jnp.zeros((L,), jnp.int32)
        for e in range(1, NE):
            gt = logits[e] > b1l
            b1l = jnp.where(gt, logits[e], b1l)
            b1v = jnp.where(gt, p[e], b1v)
            b1i = jnp.where(gt, jnp.int32(e), b1i)
        b2l = jnp.full((L,), -jnp.inf, jnp.float32)
        b2v = jnp.zeros((L,), jnp.float32)
        b2i = jnp.zeros((L,), jnp.int32)
        for e in range(NE):
            gt = jnp.logical_and(logits[e] > b2l, b1i != e)
            b2l = jnp.where(gt, logits[e], b2l)
            b2v = jnp.where(gt, p[e], b2v)
            b2i = jnp.where(gt, jnp.int32(e), b2i)

        rt = 1.0 / (b1v + b2v)
        kb = lane_k + g * (L * TOPK)
        plsc.store_scatter(tv_v, [kb], b1v * rt)
        plsc.store_scatter(tv_v, [kb + 1], b2v * rt)
        plsc.store_scatter(ti_v, [kb], b1i)
        plsc.store_scatter(ti_v, [kb + 1], b2i)
        return carry

    lax.fori_loop(0, GROUPS, body, 0)

    pltpu.sync_copy(p_v, probs_hbm.at[pl.ds(base * NE, CHUNK * NE)])
    pltpu.sync_copy(tv_v, tv_hbm.at[pl.ds(base * TOPK, CHUNK * TOPK)])
    pltpu.sync_copy(ti_v, ti_hbm.at[pl.ds(base * TOPK, CHUNK * TOPK)])


_router = pl.kernel(
    _router_body,
    out_type=_OUT_TYPE,
    mesh=_mesh,
    scratch_types=_SCRATCH,
    compiler_params=pltpu.CompilerParams(needs_layout_passes=False),
)


def kernel(hidden_states, weight):
    xb = hidden_states.astype(jnp.bfloat16)
    x_packed = lax.bitcast_convert_type(
        xb.reshape(NUM_TOKENS * HPAIR, 2), jnp.int32
    )
    w_bits = lax.bitcast_convert_type(weight, jnp.int32)
    w_bits = (w_bits + 0x7FFF + ((w_bits >> 16) & 1)) & jnp.int32(-65536)
    w_rounded = lax.bitcast_convert_type(w_bits, jnp.float32)
    probs, tv, ti = _router(x_packed, w_rounded.reshape(NE * HID))
    return (
        probs.reshape(NUM_TOKENS, NE),
        tv.reshape(NUM_TOKENS, TOPK),
        ti.reshape(NUM_TOKENS, TOPK),
    )


# speedup vs baseline: 10.9420x; 10.9420x over previous
import jax
import jax.numpy as jnp
from jax import lax
from jax.experimental import pallas as pl
from jax.experimental.pallas import tpu as pltpu
from jax.experimental.pallas import tpu_sc as plsc

NUM_TOKENS = 32768
HID = 32
NE = 4
TOPK = 2

NC = 2
NS = 16
L = 16
NW = NC * NS
CHUNK = NUM_TOKENS // NW
GROUPS = CHUNK // L

_mesh = plsc.VectorSubcoreMesh(
    core_axis_name="c", subcore_axis_name="s", num_cores=NC, num_subcores=NS
)

_OUT_TYPE = (
    jax.ShapeDtypeStruct((NE, NUM_TOKENS), jnp.float32),
    jax.ShapeDtypeStruct((TOPK, NUM_TOKENS), jnp.float32),
    jax.ShapeDtypeStruct((TOPK, NUM_TOKENS), jnp.int32),
)
_SCRATCH = [
    pltpu.VMEM((HID, CHUNK), jnp.float32),
    pltpu.VMEM((NE, HID), jnp.float32),
    pltpu.VMEM((NE, CHUNK), jnp.float32),
    pltpu.VMEM((TOPK, CHUNK), jnp.float32),
    pltpu.VMEM((TOPK, CHUNK), jnp.int32),
]


def _rtne_bf16(v):
    b = plsc.bitcast(v, jnp.int32)
    b = (b + 0x7FFF + ((b >> 16) & 1)) & jnp.int32(-65536)
    return plsc.bitcast(b, jnp.float32)


def _tree_sum(vs):
    while len(vs) > 1:
        vs = [vs[i] + vs[i + 1] for i in range(0, len(vs) - 1, 2)] + (
            [vs[-1]] if len(vs) % 2 else []
        )
    return vs[0]


def _router_body(xt_hbm, w_hbm, pt_hbm, tvt_hbm, tit_hbm, x_v, w_v, p_v, tv_v, ti_v):
    wid = lax.axis_index("s") * NC + lax.axis_index("c")
    base = wid * CHUNK

    pltpu.sync_copy(w_hbm, w_v)
    pltpu.sync_copy(xt_hbm.at[:, pl.ds(base, CHUNK)], x_v)

    w = []
    for e in range(NE):
        halves = [
            _rtne_bf16(w_v[e, pl.ds(0, L)]),
            _rtne_bf16(w_v[e, pl.ds(L, L)]),
        ]
        w.append([halves[d // L][d % L] for d in range(HID)])

    def body(g, carry):
        sl = pl.ds(g * L, L)
        cols = [_rtne_bf16(x_v[d, sl]) for d in range(HID)]

        logits = [_tree_sum([cols[d] * w[e][d] for d in range(HID)]) for e in range(NE)]

        m = jnp.maximum(
            jnp.maximum(logits[0], logits[1]), jnp.maximum(logits[2], logits[3])
        )
        ex = [jnp.exp(a - m) for a in logits]
        r = 1.0 / _tree_sum(list(ex))
        p = [e_ * r for e_ in ex]

        for e in range(NE):
            p_v[e, sl] = p[e]

        b1l, b1v, b1i = logits[0], p[0], jnp.zeros((L,), jnp.int32)
        for e in range(1, NE):
            gt = logits[e] > b1l
            b1l = jnp.where(gt, logits[e], b1l)
            b1v = jnp.where(gt, p[e], b1v)
            b1i = jnp.where(gt, jnp.int32(e), b1i)
        b2l = jnp.full((L,), -jnp.inf, jnp.float32)
        b2v = jnp.zeros((L,), jnp.float32)
        b2i = jnp.zeros((L,), jnp.int32)
        for e in range(NE):
            gt = jnp.logical_and(logits[e] > b2l, b1i != e)
            b2l = jnp.where(gt, logits[e], b2l)
            b2v = jnp.where(gt, p[e], b2v)
            b2i = jnp.where(gt, jnp.int32(e), b2i)

        rt = 1.0 / (b1v + b2v)
        tv_v[0, sl] = b1v * rt
        tv_v[1, sl] = b2v * rt
        ti_v[0, sl] = b1i
        ti_v[1, sl] = b2i
        return carry

    lax.fori_loop(0, GROUPS, body, 0)

    pltpu.sync_copy(p_v, pt_hbm.at[:, pl.ds(base, CHUNK)])
    pltpu.sync_copy(tv_v, tvt_hbm.at[:, pl.ds(base, CHUNK)])
    pltpu.sync_copy(ti_v, tit_hbm.at[:, pl.ds(base, CHUNK)])


_router = pl.kernel(
    _router_body,
    out_type=_OUT_TYPE,
    mesh=_mesh,
    scratch_types=_SCRATCH,
    compiler_params=pltpu.CompilerParams(needs_layout_passes=False),
)


def kernel(hidden_states, weight):
    pt, tvt, tit = _router(hidden_states.T, weight)
    return (pt.T, tvt.T, tit.T)
